# Initial kernel scaffold; baseline (speedup 1.0000x reference)
import jax
import jax.numpy as jnp
from jax.experimental import pallas as pl


def kernel(xyz1, xyz2, points1, points2, W0, b0, gamma0, beta0, W1, b1, gamma1, beta1):
    raise NotImplementedError("write your pallas kernel here")


# speedup vs baseline: 21.5845x; 21.5845x over previous
import jax
import jax.numpy as jnp
from jax.experimental import pallas as pl


def _stage1(xyz1_r, xyz2t_r, p1_r, p2_r, w0a_r, w0b_r, y1_r, s_r, ss_r):
    S = xyz2t_r.shape[1]
    nb = xyz1_r.shape[2]
    q = xyz1_r[0]
    k = xyz2t_r[0]

    cross = jax.lax.dot_general(k.astype(jnp.bfloat16), q.astype(jnp.bfloat16),
                                (((1,), (0,)), ((), ())),
                                preferred_element_type=jnp.float32)
    q2 = jnp.sum(q * q, axis=0, keepdims=True)
    k2 = jnp.sum(k * k, axis=1, keepdims=True)
    d = (q2 + k2) - 2.0 * cross

    iota = jax.lax.broadcasted_iota(jnp.int32, (S, nb), 0)
    dmins, ohs = [], []
    for _ in range(3):
        dmin = jnp.min(d, axis=0, keepdims=True)
        amin = jnp.min(jnp.where(d == dmin, iota, S), axis=0,
                       keepdims=True)
        oh = iota == amin
        dmins.append(dmin)
        ohs.append(oh)
        d = jnp.where(oh, jnp.float32(jnp.inf), d)

    w = [1.0 / jnp.maximum(dm, 1e-10) for dm in dmins]
    wtot = w[0] + w[1] + w[2]
    zero = jnp.float32(0)
    A = (jnp.where(ohs[0], w[0] / wtot, zero)
         + jnp.where(ohs[1], w[1] / wtot, zero)
         + jnp.where(ohs[2], w[2] / wtot, zero))

    interp = jax.lax.dot_general(p2_r[0], A, (((1,), (0,)), ((), ())),
                                 preferred_element_type=jnp.float32)
    y = (jax.lax.dot_general(w0a_r[...], p1_r[0], (((1,), (0,)), ((), ())),
                             preferred_element_type=jnp.float32)
         + jax.lax.dot_general(w0b_r[...], interp, (((1,), (0,)), ((), ())),
                               preferred_element_type=jnp.float32))
    y1_r[0] = y

    @pl.when(jnp.logical_and(pl.program_id(0) == 0, pl.program_id(1) == 0))
    def _init():
        s_r[...] = jnp.zeros_like(s_r)
        ss_r[...] = jnp.zeros_like(ss_r)

    s_r[...] += jnp.sum(y, axis=1, keepdims=True)
    ss_r[...] += jnp.sum(y * y, axis=1, keepdims=True)


def _stage2(y1_r, a1_r, c1_r, w1_r, y2_r, s_r, ss_r):
    h = jnp.maximum(a1_r[...] * y1_r[0] + c1_r[...], jnp.float32(0))
    y = jax.lax.dot_general(w1_r[...], h, (((1,), (0,)), ((), ())),
                            preferred_element_type=jnp.float32)
    y2_r[0] = y

    @pl.when(jnp.logical_and(pl.program_id(0) == 0, pl.program_id(1) == 0))
    def _init():
        s_r[...] = jnp.zeros_like(s_r)
        ss_r[...] = jnp.zeros_like(ss_r)

    s_r[...] += jnp.sum(y, axis=1, keepdims=True)
    ss_r[...] += jnp.sum(y * y, axis=1, keepdims=True)


def _stage3(y2_r, a2_r, c2_r, out_r):
    out_r[0] = jnp.maximum(a2_r[...] * y2_r[0] + c2_r[...], jnp.float32(0))


def kernel(xyz1, xyz2, points1, points2, W0, b0, gamma0, beta0,
           W1, b1, gamma1, beta1):
    B, _, N = xyz1.shape
    S = xyz2.shape[2]
    D = points1.shape[1]
    O0 = W0.shape[0]
    O1 = W1.shape[0]
    NB = 512
    nblk = N // NB
    cnt = jnp.float32(B * N)

    xyz2t = jnp.transpose(xyz2, (0, 2, 1))
    w0a = W0[:, :D]
    w0b = W0[:, D:]

    y1, s1, ss1 = pl.pallas_call(
        _stage1,
        grid=(B, nblk),
        in_specs=[
            pl.BlockSpec((1, 3, NB), lambda b, n: (b, 0, n)),
            pl.BlockSpec((1, S, 3), lambda b, n: (b, 0, 0)),
            pl.BlockSpec((1, D, NB), lambda b, n: (b, 0, n)),
            pl.BlockSpec((1, D, S), lambda b, n: (b, 0, 0)),
            pl.BlockSpec((O0, D), lambda b, n: (0, 0)),
            pl.BlockSpec((O0, D), lambda b, n: (0, 0)),
        ],
        out_specs=[
            pl.BlockSpec((1, O0, NB), lambda b, n: (b, 0, n)),
            pl.BlockSpec((O0, 1), lambda b, n: (0, 0)),
            pl.BlockSpec((O0, 1), lambda b, n: (0, 0)),
        ],
        out_shape=[
            jax.ShapeDtypeStruct((B, O0, N), jnp.float32),
            jax.ShapeDtypeStruct((O0, 1), jnp.float32),
            jax.ShapeDtypeStruct((O0, 1), jnp.float32),
        ],
    )(xyz1, xyz2t, points1, points2, w0a, w0b)

    mean1 = s1 / cnt
    var1 = ss1 / cnt - mean1 * mean1
    a1 = gamma0[:, None] * jax.lax.rsqrt(var1 + 1e-5)
    c1 = beta0[:, None] - mean1 * a1

    y2, s2, ss2 = pl.pallas_call(
        _stage2,
        grid=(B, nblk),
        in_specs=[
            pl.BlockSpec((1, O0, NB), lambda b, n: (b, 0, n)),
            pl.BlockSpec((O0, 1), lambda b, n: (0, 0)),
            pl.BlockSpec((O0, 1), lambda b, n: (0, 0)),
            pl.BlockSpec((O1, O0), lambda b, n: (0, 0)),
        ],
        out_specs=[
            pl.BlockSpec((1, O1, NB), lambda b, n: (b, 0, n)),
            pl.BlockSpec((O1, 1), lambda b, n: (0, 0)),
            pl.BlockSpec((O1, 1), lambda b, n: (0, 0)),
        ],
        out_shape=[
            jax.ShapeDtypeStruct((B, O1, N), jnp.float32),
            jax.ShapeDtypeStruct((O1, 1), jnp.float32),
            jax.ShapeDtypeStruct((O1, 1), jnp.float32),
        ],
    )(y1, a1, c1, W1)

    mean2 = s2 / cnt
    var2 = ss2 / cnt - mean2 * mean2
    a2 = gamma1[:, None] * jax.lax.rsqrt(var2 + 1e-5)
    c2 = beta1[:, None] - mean2 * a2

    out = pl.pallas_call(
        _stage3,
        grid=(B, nblk),
        in_specs=[
            pl.BlockSpec((1, O1, NB), lambda b, n: (b, 0, n)),
            pl.BlockSpec((O1, 1), lambda b, n: (0, 0)),
            pl.BlockSpec((O1, 1), lambda b, n: (0, 0)),
        ],
        out_specs=pl.BlockSpec((1, O1, NB), lambda b, n: (b, 0, n)),
        out_shape=jax.ShapeDtypeStruct((B, O1, N), jnp.float32),
    )(y2, a2, c2)

    return out


# speedup vs baseline: 25.8730x; 1.1987x over previous
import jax
import jax.numpy as jnp
from jax.experimental import pallas as pl


def _stage1(xyz1_r, xyz2t_r, p1_r, p2_r, w0a_r, w0b_r, y1_r, s_r, ss_r):
    S = xyz2t_r.shape[1]
    nb = xyz1_r.shape[2]
    q = xyz1_r[0]
    k = xyz2t_r[0]

    cross = jax.lax.dot_general(k.astype(jnp.bfloat16), q.astype(jnp.bfloat16),
                                (((1,), (0,)), ((), ())),
                                preferred_element_type=jnp.float32)
    q2 = jnp.sum(q * q, axis=0, keepdims=True)
    k2 = jnp.sum(k * k, axis=1, keepdims=True)
    d = (q2 + k2) - 2.0 * cross

    inf = jnp.float32(jnp.inf)
    U = jnp.zeros_like(d)
    usum = jnp.zeros((1, q.shape[1]), jnp.float32)
    for it in range(3):
        dmin = jnp.min(d, axis=0, keepdims=True)
        u = 1.0 / jnp.maximum(dmin, 1e-10)
        oh = d == dmin
        U = jnp.where(oh, u, U)
        usum = usum + u
        if it < 2:
            d = jnp.where(oh, inf, d)

    interp = jax.lax.dot_general(p2_r[0], U, (((1,), (0,)), ((), ())),
                                 preferred_element_type=jnp.float32)
    interp = interp * (1.0 / usum)
    y = (jax.lax.dot_general(w0a_r[...], p1_r[0], (((1,), (0,)), ((), ())),
                             preferred_element_type=jnp.float32)
         + jax.lax.dot_general(w0b_r[...], interp, (((1,), (0,)), ((), ())),
                               preferred_element_type=jnp.float32))
    y1_r[0] = y

    @pl.when(jnp.logical_and(pl.program_id(0) == 0, pl.program_id(1) == 0))
    def _init():
        s_r[...] = jnp.zeros_like(s_r)
        ss_r[...] = jnp.zeros_like(ss_r)

    s_r[...] += jnp.sum(y, axis=1, keepdims=True)
    ss_r[...] += jnp.sum(y * y, axis=1, keepdims=True)


def _stage2(y1_r, a1_r, c1_r, w1_r, y2_r, s_r, ss_r):
    h = jnp.maximum(a1_r[...] * y1_r[0] + c1_r[...], jnp.float32(0))
    y = jax.lax.dot_general(w1_r[...], h, (((1,), (0,)), ((), ())),
                            preferred_element_type=jnp.float32)
    y2_r[0] = y

    @pl.when(jnp.logical_and(pl.program_id(0) == 0, pl.program_id(1) == 0))
    def _init():
        s_r[...] = jnp.zeros_like(s_r)
        ss_r[...] = jnp.zeros_like(ss_r)

    s_r[...] += jnp.sum(y, axis=1, keepdims=True)
    ss_r[...] += jnp.sum(y * y, axis=1, keepdims=True)


def _stage3(y2_r, a2_r, c2_r, out_r):
    out_r[0] = jnp.maximum(a2_r[...] * y2_r[0] + c2_r[...], jnp.float32(0))


def kernel(xyz1, xyz2, points1, points2, W0, b0, gamma0, beta0,
           W1, b1, gamma1, beta1):
    B, _, N = xyz1.shape
    S = xyz2.shape[2]
    D = points1.shape[1]
    O0 = W0.shape[0]
    O1 = W1.shape[0]
    NB = 512
    nblk = N // NB
    cnt = jnp.float32(B * N)

    xyz2t = jnp.transpose(xyz2, (0, 2, 1))
    w0a = W0[:, :D]
    w0b = W0[:, D:]

    y1, s1, ss1 = pl.pallas_call(
        _stage1,
        grid=(B, nblk),
        in_specs=[
            pl.BlockSpec((1, 3, NB), lambda b, n: (b, 0, n)),
            pl.BlockSpec((1, S, 3), lambda b, n: (b, 0, 0)),
            pl.BlockSpec((1, D, NB), lambda b, n: (b, 0, n)),
            pl.BlockSpec((1, D, S), lambda b, n: (b, 0, 0)),
            pl.BlockSpec((O0, D), lambda b, n: (0, 0)),
            pl.BlockSpec((O0, D), lambda b, n: (0, 0)),
        ],
        out_specs=[
            pl.BlockSpec((1, O0, NB), lambda b, n: (b, 0, n)),
            pl.BlockSpec((O0, 1), lambda b, n: (0, 0)),
            pl.BlockSpec((O0, 1), lambda b, n: (0, 0)),
        ],
        out_shape=[
            jax.ShapeDtypeStruct((B, O0, N), jnp.float32),
            jax.ShapeDtypeStruct((O0, 1), jnp.float32),
            jax.ShapeDtypeStruct((O0, 1), jnp.float32),
        ],
    )(xyz1, xyz2t, points1, points2, w0a, w0b)

    mean1 = s1 / cnt
    var1 = ss1 / cnt - mean1 * mean1
    a1 = gamma0[:, None] * jax.lax.rsqrt(var1 + 1e-5)
    c1 = beta0[:, None] - mean1 * a1

    y2, s2, ss2 = pl.pallas_call(
        _stage2,
        grid=(B, nblk),
        in_specs=[
            pl.BlockSpec((1, O0, NB), lambda b, n: (b, 0, n)),
            pl.BlockSpec((O0, 1), lambda b, n: (0, 0)),
            pl.BlockSpec((O0, 1), lambda b, n: (0, 0)),
            pl.BlockSpec((O1, O0), lambda b, n: (0, 0)),
        ],
        out_specs=[
            pl.BlockSpec((1, O1, NB), lambda b, n: (b, 0, n)),
            pl.BlockSpec((O1, 1), lambda b, n: (0, 0)),
            pl.BlockSpec((O1, 1), lambda b, n: (0, 0)),
        ],
        out_shape=[
            jax.ShapeDtypeStruct((B, O1, N), jnp.float32),
            jax.ShapeDtypeStruct((O1, 1), jnp.float32),
            jax.ShapeDtypeStruct((O1, 1), jnp.float32),
        ],
    )(y1, a1, c1, W1)

    mean2 = s2 / cnt
    var2 = ss2 / cnt - mean2 * mean2
    a2 = gamma1[:, None] * jax.lax.rsqrt(var2 + 1e-5)
    c2 = beta1[:, None] - mean2 * a2

    out = pl.pallas_call(
        _stage3,
        grid=(B, nblk),
        in_specs=[
            pl.BlockSpec((1, O1, NB), lambda b, n: (b, 0, n)),
            pl.BlockSpec((O1, 1), lambda b, n: (0, 0)),
            pl.BlockSpec((O1, 1), lambda b, n: (0, 0)),
        ],
        out_specs=pl.BlockSpec((1, O1, NB), lambda b, n: (b, 0, n)),
        out_shape=jax.ShapeDtypeStruct((B, O1, N), jnp.float32),
    )(y2, a2, c2)

    return out


# speedup vs baseline: 37.0148x; 1.4306x over previous
import jax
import jax.numpy as jnp
from jax.experimental import pallas as pl


def _stage1(xyz1_r, xyz2t_r, p1_r, p2_r, w0a_r, w0b_r, y1_r, s_r, ss_r):
    q = xyz1_r[0]
    k = xyz2t_r[0]

    cross = jax.lax.dot_general(k.astype(jnp.bfloat16), q.astype(jnp.bfloat16),
                                (((1,), (0,)), ((), ())),
                                preferred_element_type=jnp.float32)
    q2 = jnp.sum(q * q, axis=0, keepdims=True)
    k2 = jnp.sum(k * k, axis=1, keepdims=True)
    d = (q2 + k2) - 2.0 * cross

    inf = jnp.float32(jnp.inf)
    U = jnp.zeros_like(d)
    usum = jnp.zeros((1, q.shape[1]), jnp.float32)
    for it in range(3):
        dmin = jnp.min(d, axis=0, keepdims=True)
        u = 1.0 / jnp.maximum(dmin, 1e-10)
        oh = d == dmin
        U = jnp.where(oh, u, U)
        usum = usum + u
        if it < 2:
            d = jnp.where(oh, inf, d)

    interp = jax.lax.dot_general(p2_r[0], U, (((1,), (0,)), ((), ())),
                                 preferred_element_type=jnp.float32)
    interp = interp * (1.0 / usum)

    y = (jax.lax.dot_general(w0a_r[...], p1_r[0], (((1,), (0,)), ((), ())),
                             preferred_element_type=jnp.float32)
         + jax.lax.dot_general(w0b_r[...], interp, (((1,), (0,)), ((), ())),
                               preferred_element_type=jnp.float32))
    y1_r[...] = y

    @pl.when(jnp.logical_and(pl.program_id(0) == 0, pl.program_id(1) == 0))
    def _init():
        s_r[...] = jnp.zeros_like(s_r)
        ss_r[...] = jnp.zeros_like(ss_r)

    s_r[...] += jnp.sum(y, axis=1, keepdims=True)
    ss_r[...] += jnp.sum(y * y, axis=1, keepdims=True)


def _stage2(y1_r, a1_r, c1_r, w1_r, y2_r, s_r, ss_r):
    h = jnp.maximum(a1_r[...] * y1_r[...] + c1_r[...], jnp.float32(0))
    y = jax.lax.dot_general(w1_r[...], h, (((1,), (0,)), ((), ())),
                            preferred_element_type=jnp.float32)
    y2_r[...] = y

    @pl.when(pl.program_id(0) == 0)
    def _init():
        s_r[...] = jnp.zeros_like(s_r)
        ss_r[...] = jnp.zeros_like(ss_r)

    s_r[...] += jnp.sum(y, axis=1, keepdims=True)
    ss_r[...] += jnp.sum(y * y, axis=1, keepdims=True)


def _stage3(y2_r, a2_r, c2_r, out_r):
    out_r[0] = jnp.maximum(a2_r[...] * y2_r[...] + c2_r[...], jnp.float32(0))


def kernel(xyz1, xyz2, points1, points2, W0, b0, gamma0, beta0,
           W1, b1, gamma1, beta1):
    B, _, N = xyz1.shape
    S = xyz2.shape[2]
    D = points1.shape[1]
    O0 = W0.shape[0]
    O1 = W1.shape[0]
    BN = B * N
    NB1 = 1024
    nblk1 = N // NB1
    NB2 = 2048
    NB3 = 2048
    nblk3 = N // NB3
    cnt = jnp.float32(BN)

    xyz2t = jnp.transpose(xyz2, (0, 2, 1))
    w0a = W0[:, :D]
    w0b = W0[:, D:]

    y1, s1, ss1 = pl.pallas_call(
        _stage1,
        grid=(B, nblk1),
        in_specs=[
            pl.BlockSpec((1, 3, NB1), lambda b, n: (b, 0, n)),
            pl.BlockSpec((1, S, 3), lambda b, n: (b, 0, 0)),
            pl.BlockSpec((1, D, NB1), lambda b, n: (b, 0, n)),
            pl.BlockSpec((1, D, S), lambda b, n: (b, 0, 0)),
            pl.BlockSpec((O0, D), lambda b, n: (0, 0)),
            pl.BlockSpec((O0, D), lambda b, n: (0, 0)),
        ],
        out_specs=[
            pl.BlockSpec((O0, NB1), lambda b, n: (0, b * (N // NB1) + n)),
            pl.BlockSpec((O0, 1), lambda b, n: (0, 0)),
            pl.BlockSpec((O0, 1), lambda b, n: (0, 0)),
        ],
        out_shape=[
            jax.ShapeDtypeStruct((O0, BN), jnp.float32),
            jax.ShapeDtypeStruct((O0, 1), jnp.float32),
            jax.ShapeDtypeStruct((O0, 1), jnp.float32),
        ],
    )(xyz1, xyz2t, points1, points2, w0a, w0b)

    mean1 = s1 / cnt
    var1 = ss1 / cnt - mean1 * mean1
    a1 = gamma0[:, None] * jax.lax.rsqrt(var1 + 1e-5)
    c1 = beta0[:, None] - mean1 * a1

    y2, s2, ss2 = pl.pallas_call(
        _stage2,
        grid=(BN // NB2,),
        in_specs=[
            pl.BlockSpec((O0, NB2), lambda i: (0, i)),
            pl.BlockSpec((O0, 1), lambda i: (0, 0)),
            pl.BlockSpec((O0, 1), lambda i: (0, 0)),
            pl.BlockSpec((O1, O0), lambda i: (0, 0)),
        ],
        out_specs=[
            pl.BlockSpec((O1, NB2), lambda i: (0, i)),
            pl.BlockSpec((O1, 1), lambda i: (0, 0)),
            pl.BlockSpec((O1, 1), lambda i: (0, 0)),
        ],
        out_shape=[
            jax.ShapeDtypeStruct((O1, BN), jnp.float32),
            jax.ShapeDtypeStruct((O1, 1), jnp.float32),
            jax.ShapeDtypeStruct((O1, 1), jnp.float32),
        ],
    )(y1, a1, c1, W1)

    mean2 = s2 / cnt
    var2 = ss2 / cnt - mean2 * mean2
    a2 = gamma1[:, None] * jax.lax.rsqrt(var2 + 1e-5)
    c2 = beta1[:, None] - mean2 * a2

    out = pl.pallas_call(
        _stage3,
        grid=(B * nblk3,),
        in_specs=[
            pl.BlockSpec((O1, NB3), lambda i: (0, i)),
            pl.BlockSpec((O1, 1), lambda i: (0, 0)),
            pl.BlockSpec((O1, 1), lambda i: (0, 0)),
        ],
        out_specs=pl.BlockSpec((1, O1, NB3),
                               lambda i: (i // (N // NB3), 0, i % (N // NB3))),
        out_shape=jax.ShapeDtypeStruct((B, O1, N), jnp.float32),
    )(y2, a2, c2)

    return out


# speedup vs baseline: 41.2513x; 1.1145x over previous
import functools

import jax
import jax.numpy as jnp
from jax.experimental import pallas as pl


def _stage1(xyz1_r, xyz2t_r, p1_r, p2_r, w0a_r, w0b_r, y1_r, s_r, ss_r):
    q = xyz1_r[0]
    k = xyz2t_r[0]

    cross = jax.lax.dot_general(k.astype(jnp.bfloat16), q.astype(jnp.bfloat16),
                                (((1,), (0,)), ((), ())),
                                preferred_element_type=jnp.float32)
    q2 = jnp.sum(q * q, axis=0, keepdims=True)
    k2 = jnp.sum(k * k, axis=1, keepdims=True)
    d = (q2 + k2) - 2.0 * cross

    inf = jnp.float32(jnp.inf)
    U = jnp.zeros_like(d)
    usum = jnp.zeros((1, q.shape[1]), jnp.float32)
    for it in range(3):
        dmin = jnp.min(d, axis=0, keepdims=True)
        u = 1.0 / jnp.maximum(dmin, 1e-10)
        oh = d == dmin
        U = jnp.where(oh, u, U)
        usum = usum + u
        if it < 2:
            d = jnp.where(oh, inf, d)

    interp = jax.lax.dot_general(p2_r[0], U, (((1,), (0,)), ((), ())),
                                 preferred_element_type=jnp.float32)
    interp = interp * (1.0 / usum)

    y = (jax.lax.dot_general(w0a_r[...], p1_r[0], (((1,), (0,)), ((), ())),
                             preferred_element_type=jnp.float32)
         + jax.lax.dot_general(w0b_r[...], interp, (((1,), (0,)), ((), ())),
                               preferred_element_type=jnp.float32))
    y1_r[...] = y.astype(jnp.bfloat16)

    @pl.when(jnp.logical_and(pl.program_id(0) == 0, pl.program_id(1) == 0))
    def _init():
        s_r[...] = jnp.zeros_like(s_r)
        ss_r[...] = jnp.zeros_like(ss_r)

    s_r[...] += jnp.sum(y, axis=1, keepdims=True)
    ss_r[...] += jnp.sum(y * y, axis=1, keepdims=True)


def _bn_affine(s_r, ss_r, g_r, b_r, cnt):
    mean = s_r[...] * (1.0 / cnt)
    var = ss_r[...] * (1.0 / cnt) - mean * mean
    a = g_r[...] * jax.lax.rsqrt(var + 1e-5)
    c = b_r[...] - mean * a
    return a, c


def _stage2(y1_r, s1_r, ss1_r, g0_r, b0_r, w1_r, y2_r, s_r, ss_r, *, cnt):
    a1, c1 = _bn_affine(s1_r, ss1_r, g0_r, b0_r, cnt)
    h = jnp.maximum(a1 * y1_r[...].astype(jnp.float32) + c1, jnp.float32(0))
    y = jax.lax.dot_general(w1_r[...], h, (((1,), (0,)), ((), ())),
                            preferred_element_type=jnp.float32)
    y2_r[...] = y.astype(jnp.bfloat16)

    @pl.when(pl.program_id(0) == 0)
    def _init():
        s_r[...] = jnp.zeros_like(s_r)
        ss_r[...] = jnp.zeros_like(ss_r)

    s_r[...] += jnp.sum(y, axis=1, keepdims=True)
    ss_r[...] += jnp.sum(y * y, axis=1, keepdims=True)


def _stage3(y2_r, s2_r, ss2_r, g1_r, b1_r, out_r, *, cnt):
    a2, c2 = _bn_affine(s2_r, ss2_r, g1_r, b1_r, cnt)
    out_r[0] = jnp.maximum(a2 * y2_r[...].astype(jnp.float32) + c2,
                           jnp.float32(0))


def kernel(xyz1, xyz2, points1, points2, W0, b0, gamma0, beta0,
           W1, b1, gamma1, beta1):
    B, _, N = xyz1.shape
    S = xyz2.shape[2]
    D = points1.shape[1]
    O0 = W0.shape[0]
    O1 = W1.shape[0]
    BN = B * N
    NB1 = 1024
    nblk1 = N // NB1
    NB2 = 2048
    NB3 = 2048
    nblk3 = N // NB3
    cnt = float(BN)

    xyz2t = jnp.transpose(xyz2, (0, 2, 1))
    g0 = gamma0[:, None]
    be0 = beta0[:, None]
    g1 = gamma1[:, None]
    be1 = beta1[:, None]

    y1, s1, ss1 = pl.pallas_call(
        _stage1,
        grid=(B, nblk1),
        in_specs=[
            pl.BlockSpec((1, 3, NB1), lambda b, n: (b, 0, n)),
            pl.BlockSpec((1, S, 3), lambda b, n: (b, 0, 0)),
            pl.BlockSpec((1, D, NB1), lambda b, n: (b, 0, n)),
            pl.BlockSpec((1, D, S), lambda b, n: (b, 0, 0)),
            pl.BlockSpec((O0, D), lambda b, n: (0, 0)),
            pl.BlockSpec((O0, D), lambda b, n: (0, 1)),
        ],
        out_specs=[
            pl.BlockSpec((O0, NB1), lambda b, n: (0, b * (N // NB1) + n)),
            pl.BlockSpec((O0, 1), lambda b, n: (0, 0)),
            pl.BlockSpec((O0, 1), lambda b, n: (0, 0)),
        ],
        out_shape=[
            jax.ShapeDtypeStruct((O0, BN), jnp.bfloat16),
            jax.ShapeDtypeStruct((O0, 1), jnp.float32),
            jax.ShapeDtypeStruct((O0, 1), jnp.float32),
        ],
    )(xyz1, xyz2t, points1, points2, W0, W0)

    y2, s2, ss2 = pl.pallas_call(
        functools.partial(_stage2, cnt=cnt),
        grid=(BN // NB2,),
        in_specs=[
            pl.BlockSpec((O0, NB2), lambda i: (0, i)),
            pl.BlockSpec((O0, 1), lambda i: (0, 0)),
            pl.BlockSpec((O0, 1), lambda i: (0, 0)),
            pl.BlockSpec((O0, 1), lambda i: (0, 0)),
            pl.BlockSpec((O0, 1), lambda i: (0, 0)),
            pl.BlockSpec((O1, O0), lambda i: (0, 0)),
        ],
        out_specs=[
            pl.BlockSpec((O1, NB2), lambda i: (0, i)),
            pl.BlockSpec((O1, 1), lambda i: (0, 0)),
            pl.BlockSpec((O1, 1), lambda i: (0, 0)),
        ],
        out_shape=[
            jax.ShapeDtypeStruct((O1, BN), jnp.bfloat16),
            jax.ShapeDtypeStruct((O1, 1), jnp.float32),
            jax.ShapeDtypeStruct((O1, 1), jnp.float32),
        ],
    )(y1, s1, ss1, g0, be0, W1)

    out = pl.pallas_call(
        functools.partial(_stage3, cnt=cnt),
        grid=(B * nblk3,),
        in_specs=[
            pl.BlockSpec((O1, NB3), lambda i: (0, i)),
            pl.BlockSpec((O1, 1), lambda i: (0, 0)),
            pl.BlockSpec((O1, 1), lambda i: (0, 0)),
            pl.BlockSpec((O1, 1), lambda i: (0, 0)),
            pl.BlockSpec((O1, 1), lambda i: (0, 0)),
        ],
        out_specs=pl.BlockSpec((1, O1, NB3),
                               lambda i: (i // (N // NB3), 0, i % (N // NB3))),
        out_shape=jax.ShapeDtypeStruct((B, O1, N), jnp.float32),
    )(y2, s2, ss2, g1, be1)

    return out


# speedup vs baseline: 44.6719x; 1.0829x over previous
import functools

import jax
import jax.numpy as jnp
from jax.experimental import pallas as pl


def _stage1(xyz1_r, xyz2t_r, p1_r, p2_r, w0a_r, w0b_r, y1_r, s_r, ss_r):
    q = xyz1_r[0]
    k = xyz2t_r[0]

    S = k.shape[0]
    nb = q.shape[1]

    cross = jax.lax.dot_general(k.astype(jnp.bfloat16), q.astype(jnp.bfloat16),
                                (((1,), (0,)), ((), ())),
                                preferred_element_type=jnp.float32)
    q2 = jnp.sum(q * q, axis=0, keepdims=True)
    k2 = jnp.sum(k * k, axis=1, keepdims=True)
    d = k2 - 2.0 * cross

    def _insert3(r1, r2, r3, x):
        a = jnp.minimum(r1, x)
        b = jnp.maximum(r1, x)
        c = jnp.minimum(r2, b)
        e = jnp.maximum(r2, b)
        f = jnp.minimum(r3, e)
        return a, c, f

    inf8 = jnp.full((8, nb), jnp.inf, jnp.float32)
    r1, r2, r3 = d[0:8], inf8, inf8
    for i in range(1, S // 8):
        r1, r2, r3 = _insert3(r1, r2, r3, d[8 * i:8 * (i + 1)])

    for sh in (4, 2, 1):
        s1 = jnp.roll(r1, sh, axis=0)
        s2 = jnp.roll(r2, sh, axis=0)
        s3 = jnp.roll(r3, sh, axis=0)
        r1, r2, r3 = _insert3(r1, r2, r3, s1)
        r1, r2, r3 = _insert3(r1, r2, r3, s2)
        r1, r2, r3 = _insert3(r1, r2, r3, s3)

    m1 = r1[0:1, :]
    m2 = r2[0:1, :]
    m3 = r3[0:1, :]
    u1 = 1.0 / jnp.maximum(m1 + q2, 1e-10)
    u2 = 1.0 / jnp.maximum(m2 + q2, 1e-10)
    u3 = 1.0 / jnp.maximum(m3 + q2, 1e-10)
    usum = u1 + u2 + u3

    zero = jnp.float32(0)
    U = jnp.where(d == m1, u1,
                  jnp.where(d == m2, u2,
                            jnp.where(d == m3, u3, zero)))

    interp = jax.lax.dot_general(p2_r[0], U, (((1,), (0,)), ((), ())),
                                 preferred_element_type=jnp.float32)
    interp = interp * (1.0 / usum)

    y = (jax.lax.dot_general(w0a_r[...], p1_r[0], (((1,), (0,)), ((), ())),
                             preferred_element_type=jnp.float32)
         + jax.lax.dot_general(w0b_r[...], interp, (((1,), (0,)), ((), ())),
                               preferred_element_type=jnp.float32))
    y1_r[...] = y.astype(jnp.bfloat16)

    @pl.when(jnp.logical_and(pl.program_id(0) == 0, pl.program_id(1) == 0))
    def _init():
        s_r[...] = jnp.zeros_like(s_r)
        ss_r[...] = jnp.zeros_like(ss_r)

    s_r[...] += jnp.sum(y, axis=1, keepdims=True)
    ss_r[...] += jnp.sum(y * y, axis=1, keepdims=True)


def _bn_affine(s_r, ss_r, g_r, b_r, cnt):
    mean = s_r[...] * (1.0 / cnt)
    var = ss_r[...] * (1.0 / cnt) - mean * mean
    a = g_r[...] * jax.lax.rsqrt(var + 1e-5)
    c = b_r[...] - mean * a
    return a, c


def _stage2(y1_r, s1_r, ss1_r, g0_r, b0_r, w1_r, y2_r, s_r, ss_r, *, cnt):
    a1, c1 = _bn_affine(s1_r, ss1_r, g0_r, b0_r, cnt)
    h = jnp.maximum(a1 * y1_r[...].astype(jnp.float32) + c1, jnp.float32(0))
    y = jax.lax.dot_general(w1_r[...], h, (((1,), (0,)), ((), ())),
                            preferred_element_type=jnp.float32)
    y2_r[...] = y.astype(jnp.bfloat16)

    @pl.when(pl.program_id(0) == 0)
    def _init():
        s_r[...] = jnp.zeros_like(s_r)
        ss_r[...] = jnp.zeros_like(ss_r)

    s_r[...] += jnp.sum(y, axis=1, keepdims=True)
    ss_r[...] += jnp.sum(y * y, axis=1, keepdims=True)


def _stage3(y2_r, s2_r, ss2_r, g1_r, b1_r, out_r, *, cnt):
    a2, c2 = _bn_affine(s2_r, ss2_r, g1_r, b1_r, cnt)
    out_r[0] = jnp.maximum(a2 * y2_r[...].astype(jnp.float32) + c2,
                           jnp.float32(0))


def kernel(xyz1, xyz2, points1, points2, W0, b0, gamma0, beta0,
           W1, b1, gamma1, beta1):
    B, _, N = xyz1.shape
    S = xyz2.shape[2]
    D = points1.shape[1]
    O0 = W0.shape[0]
    O1 = W1.shape[0]
    BN = B * N
    NB1 = 1024
    nblk1 = N // NB1
    NB2 = 2048
    NB3 = 2048
    nblk3 = N // NB3
    cnt = float(BN)

    xyz2t = jnp.transpose(xyz2, (0, 2, 1))
    g0 = gamma0[:, None]
    be0 = beta0[:, None]
    g1 = gamma1[:, None]
    be1 = beta1[:, None]

    y1, s1, ss1 = pl.pallas_call(
        _stage1,
        grid=(B, nblk1),
        in_specs=[
            pl.BlockSpec((1, 3, NB1), lambda b, n: (b, 0, n)),
            pl.BlockSpec((1, S, 3), lambda b, n: (b, 0, 0)),
            pl.BlockSpec((1, D, NB1), lambda b, n: (b, 0, n)),
            pl.BlockSpec((1, D, S), lambda b, n: (b, 0, 0)),
            pl.BlockSpec((O0, D), lambda b, n: (0, 0)),
            pl.BlockSpec((O0, D), lambda b, n: (0, 1)),
        ],
        out_specs=[
            pl.BlockSpec((O0, NB1), lambda b, n: (0, b * (N // NB1) + n)),
            pl.BlockSpec((O0, 1), lambda b, n: (0, 0)),
            pl.BlockSpec((O0, 1), lambda b, n: (0, 0)),
        ],
        out_shape=[
            jax.ShapeDtypeStruct((O0, BN), jnp.bfloat16),
            jax.ShapeDtypeStruct((O0, 1), jnp.float32),
            jax.ShapeDtypeStruct((O0, 1), jnp.float32),
        ],
    )(xyz1, xyz2t, points1, points2, W0, W0)

    y2, s2, ss2 = pl.pallas_call(
        functools.partial(_stage2, cnt=cnt),
        grid=(BN // NB2,),
        in_specs=[
            pl.BlockSpec((O0, NB2), lambda i: (0, i)),
            pl.BlockSpec((O0, 1), lambda i: (0, 0)),
            pl.BlockSpec((O0, 1), lambda i: (0, 0)),
            pl.BlockSpec((O0, 1), lambda i: (0, 0)),
            pl.BlockSpec((O0, 1), lambda i: (0, 0)),
            pl.BlockSpec((O1, O0), lambda i: (0, 0)),
        ],
        out_specs=[
            pl.BlockSpec((O1, NB2), lambda i: (0, i)),
            pl.BlockSpec((O1, 1), lambda i: (0, 0)),
            pl.BlockSpec((O1, 1), lambda i: (0, 0)),
        ],
        out_shape=[
            jax.ShapeDtypeStruct((O1, BN), jnp.bfloat16),
            jax.ShapeDtypeStruct((O1, 1), jnp.float32),
            jax.ShapeDtypeStruct((O1, 1), jnp.float32),
        ],
    )(y1, s1, ss1, g0, be0, W1)

    out = pl.pallas_call(
        functools.partial(_stage3, cnt=cnt),
        grid=(B * nblk3,),
        in_specs=[
            pl.BlockSpec((O1, NB3), lambda i: (0, i)),
            pl.BlockSpec((O1, 1), lambda i: (0, 0)),
            pl.BlockSpec((O1, 1), lambda i: (0, 0)),
            pl.BlockSpec((O1, 1), lambda i: (0, 0)),
            pl.BlockSpec((O1, 1), lambda i: (0, 0)),
        ],
        out_specs=pl.BlockSpec((1, O1, NB3),
                               lambda i: (i // (N // NB3), 0, i % (N // NB3))),
        out_shape=jax.ShapeDtypeStruct((B, O1, N), jnp.float32),
    )(y2, s2, ss2, g1, be1)

    return out


# speedup vs baseline: 48.2672x; 1.0805x over previous
import functools

import jax
import jax.numpy as jnp
from jax.experimental import pallas as pl


def _stage1(xyz1_r, xyz2t_r, p1_r, p2_r, w0a_r, w0b_r, y1_r, s_r, ss_r):
    q = xyz1_r[0]
    k = xyz2t_r[0]

    S = k.shape[0]
    nb = q.shape[1]

    cross2 = jax.lax.dot_general(k.astype(jnp.bfloat16),
                                 (q * 2.0).astype(jnp.bfloat16),
                                 (((1,), (0,)), ((), ())),
                                 preferred_element_type=jnp.float32)
    q2 = jnp.sum(q * q, axis=0, keepdims=True)
    k2 = jnp.sum(k * k, axis=1, keepdims=True)
    d = k2 - cross2

    def _insert3(r1, r2, r3, x):
        a = jnp.minimum(r1, x)
        b = jnp.maximum(r1, x)
        c = jnp.minimum(r2, b)
        e = jnp.maximum(r2, b)
        f = jnp.minimum(r3, e)
        return a, c, f

    inf8 = jnp.full((8, nb), jnp.inf, jnp.float32)
    r1, r2, r3 = d[0:8], inf8, inf8
    for i in range(1, S // 8):
        r1, r2, r3 = _insert3(r1, r2, r3, d[8 * i:8 * (i + 1)])

    for sh in (4, 2, 1):
        s1 = jnp.roll(r1, sh, axis=0)
        s2 = jnp.roll(r2, sh, axis=0)
        s3 = jnp.roll(r3, sh, axis=0)
        r1, r2, r3 = _insert3(r1, r2, r3, s1)
        r1, r2, r3 = _insert3(r1, r2, r3, s2)
        r1, r2, r3 = _insert3(r1, r2, r3, s3)

    m1 = r1[0:1, :]
    m2 = r2[0:1, :]
    m3 = r3[0:1, :]
    u1 = 1.0 / jnp.maximum(m1 + q2, 1e-10)
    u2 = 1.0 / jnp.maximum(m2 + q2, 1e-10)
    u3 = 1.0 / jnp.maximum(m3 + q2, 1e-10)
    usum = u1 + u2 + u3

    zero = jnp.float32(0)
    U = jnp.where(d == m1, u1,
                  jnp.where(d == m2, u2,
                            jnp.where(d == m3, u3, zero)))

    interp = jax.lax.dot_general(p2_r[0], U, (((1,), (0,)), ((), ())),
                                 preferred_element_type=jnp.float32)
    interp = interp * (1.0 / usum)

    y = (jax.lax.dot_general(w0a_r[...], p1_r[0], (((1,), (0,)), ((), ())),
                             preferred_element_type=jnp.float32)
         + jax.lax.dot_general(w0b_r[...], interp, (((1,), (0,)), ((), ())),
                               preferred_element_type=jnp.float32))
    y1_r[...] = y.astype(jnp.bfloat16)

    @pl.when(jnp.logical_and(pl.program_id(0) == 0, pl.program_id(1) == 0))
    def _init():
        s_r[...] = jnp.zeros_like(s_r)
        ss_r[...] = jnp.zeros_like(ss_r)

    s_r[...] += jnp.sum(y, axis=1, keepdims=True)
    ss_r[...] += jnp.sum(y * y, axis=1, keepdims=True)


def _bn_affine(s_r, ss_r, g_r, b_r, cnt):
    mean = s_r[...] * (1.0 / cnt)
    var = ss_r[...] * (1.0 / cnt) - mean * mean
    a = g_r[...] * jax.lax.rsqrt(var + 1e-5)
    c = b_r[...] - mean * a
    return a, c


def _stage2(y1_r, s1_r, ss1_r, g0_r, b0_r, w1_r, y2_r, s_r, ss_r, *, cnt):
    a1, c1 = _bn_affine(s1_r, ss1_r, g0_r, b0_r, cnt)
    h = jnp.maximum(a1 * y1_r[...].astype(jnp.float32) + c1, jnp.float32(0))
    y = jax.lax.dot_general(w1_r[...], h, (((1,), (0,)), ((), ())),
                            preferred_element_type=jnp.float32)
    y2_r[...] = y.astype(jnp.bfloat16)

    @pl.when(pl.program_id(0) == 0)
    def _init():
        s_r[...] = jnp.zeros_like(s_r)
        ss_r[...] = jnp.zeros_like(ss_r)

    s_r[...] += jnp.sum(y, axis=1, keepdims=True)
    ss_r[...] += jnp.sum(y * y, axis=1, keepdims=True)


def _stage3(y2_r, s2_r, ss2_r, g1_r, b1_r, out_r, *, cnt):
    a2, c2 = _bn_affine(s2_r, ss2_r, g1_r, b1_r, cnt)
    out_r[0] = jnp.maximum(a2 * y2_r[...].astype(jnp.float32) + c2,
                           jnp.float32(0))


def kernel(xyz1, xyz2, points1, points2, W0, b0, gamma0, beta0,
           W1, b1, gamma1, beta1):
    B, _, N = xyz1.shape
    S = xyz2.shape[2]
    D = points1.shape[1]
    O0 = W0.shape[0]
    O1 = W1.shape[0]
    BN = B * N
    NB1 = 2048
    nblk1 = N // NB1
    NB2 = 4096
    NB3 = 4096
    nblk3 = N // NB3
    cnt = float(BN)

    xyz2t = jnp.transpose(xyz2, (0, 2, 1))
    g0 = gamma0[:, None]
    be0 = beta0[:, None]
    g1 = gamma1[:, None]
    be1 = beta1[:, None]

    y1, s1, ss1 = pl.pallas_call(
        _stage1,
        grid=(B, nblk1),
        in_specs=[
            pl.BlockSpec((1, 3, NB1), lambda b, n: (b, 0, n)),
            pl.BlockSpec((1, S, 3), lambda b, n: (b, 0, 0)),
            pl.BlockSpec((1, D, NB1), lambda b, n: (b, 0, n)),
            pl.BlockSpec((1, D, S), lambda b, n: (b, 0, 0)),
            pl.BlockSpec((O0, D), lambda b, n: (0, 0)),
            pl.BlockSpec((O0, D), lambda b, n: (0, 1)),
        ],
        out_specs=[
            pl.BlockSpec((O0, NB1), lambda b, n: (0, b * (N // NB1) + n)),
            pl.BlockSpec((O0, 1), lambda b, n: (0, 0)),
            pl.BlockSpec((O0, 1), lambda b, n: (0, 0)),
        ],
        out_shape=[
            jax.ShapeDtypeStruct((O0, BN), jnp.bfloat16),
            jax.ShapeDtypeStruct((O0, 1), jnp.float32),
            jax.ShapeDtypeStruct((O0, 1), jnp.float32),
        ],
    )(xyz1, xyz2t, points1, points2, W0, W0)

    y2, s2, ss2 = pl.pallas_call(
        functools.partial(_stage2, cnt=cnt),
        grid=(BN // NB2,),
        in_specs=[
            pl.BlockSpec((O0, NB2), lambda i: (0, i)),
            pl.BlockSpec((O0, 1), lambda i: (0, 0)),
            pl.BlockSpec((O0, 1), lambda i: (0, 0)),
            pl.BlockSpec((O0, 1), lambda i: (0, 0)),
            pl.BlockSpec((O0, 1), lambda i: (0, 0)),
            pl.BlockSpec((O1, O0), lambda i: (0, 0)),
        ],
        out_specs=[
            pl.BlockSpec((O1, NB2), lambda i: (0, i)),
            pl.BlockSpec((O1, 1), lambda i: (0, 0)),
            pl.BlockSpec((O1, 1), lambda i: (0, 0)),
        ],
        out_shape=[
            jax.ShapeDtypeStruct((O1, BN), jnp.bfloat16),
            jax.ShapeDtypeStruct((O1, 1), jnp.float32),
            jax.ShapeDtypeStruct((O1, 1), jnp.float32),
        ],
    )(y1, s1, ss1, g0, be0, W1)

    out = pl.pallas_call(
        functools.partial(_stage3, cnt=cnt),
        grid=(B * nblk3,),
        in_specs=[
            pl.BlockSpec((O1, NB3), lambda i: (0, i)),
            pl.BlockSpec((O1, 1), lambda i: (0, 0)),
            pl.BlockSpec((O1, 1), lambda i: (0, 0)),
            pl.BlockSpec((O1, 1), lambda i: (0, 0)),
            pl.BlockSpec((O1, 1), lambda i: (0, 0)),
        ],
        out_specs=pl.BlockSpec((1, O1, NB3),
                               lambda i: (i // (N // NB3), 0, i % (N // NB3))),
        out_shape=jax.ShapeDtypeStruct((B, O1, N), jnp.float32),
    )(y2, s2, ss2, g1, be1)

    return out


# speedup vs baseline: 48.4675x; 1.0042x over previous
import functools

import jax
import jax.numpy as jnp
from jax.experimental import pallas as pl


def _stage1(xyz1_r, xyz2t_r, p1_r, p2_r, w0a_r, w0b_r, y1_r, s_r, ss_r):
    q = xyz1_r[0]
    k = xyz2t_r[0]

    S = k.shape[0]
    nb = q.shape[1]

    cross2 = jax.lax.dot_general(k.astype(jnp.bfloat16),
                                 (q * 2.0).astype(jnp.bfloat16),
                                 (((1,), (0,)), ((), ())),
                                 preferred_element_type=jnp.float32)
    q2 = jnp.sum(q * q, axis=0, keepdims=True)
    k2 = jnp.sum(k * k, axis=1, keepdims=True)
    d = k2 - cross2

    def _insert3(r1, r2, r3, x):
        a = jnp.minimum(r1, x)
        b = jnp.maximum(r1, x)
        c = jnp.minimum(r2, b)
        e = jnp.maximum(r2, b)
        f = jnp.minimum(r3, e)
        return a, c, f

    inf8 = jnp.full((8, nb), jnp.inf, jnp.float32)
    r1, r2, r3 = d[0:8], inf8, inf8
    for i in range(1, S // 8):
        r1, r2, r3 = _insert3(r1, r2, r3, d[8 * i:8 * (i + 1)])

    for sh in (4, 2, 1):
        s1 = jnp.roll(r1, sh, axis=0)
        s2 = jnp.roll(r2, sh, axis=0)
        s3 = jnp.roll(r3, sh, axis=0)
        r1, r2, r3 = _insert3(r1, r2, r3, s1)
        r1, r2, r3 = _insert3(r1, r2, r3, s2)
        r1, r2, r3 = _insert3(r1, r2, r3, s3)

    m1 = r1[0:1, :]
    m2 = r2[0:1, :]
    m3 = r3[0:1, :]
    u1 = 1.0 / jnp.maximum(m1 + q2, 1e-10)
    u2 = 1.0 / jnp.maximum(m2 + q2, 1e-10)
    u3 = 1.0 / jnp.maximum(m3 + q2, 1e-10)
    usum = u1 + u2 + u3

    zero = jnp.float32(0)
    U = jnp.where(d <= m3,
                  pl.reciprocal(jnp.maximum(d + q2, 1e-10), approx=True),
                  zero)

    interp = jax.lax.dot_general(p2_r[0], U, (((1,), (0,)), ((), ())),
                                 preferred_element_type=jnp.float32)
    interp = (interp * (1.0 / usum)).astype(jnp.bfloat16)

    y = (jax.lax.dot_general(w0a_r[...], p1_r[0].astype(jnp.bfloat16),
                             (((1,), (0,)), ((), ())),
                             preferred_element_type=jnp.float32)
         + jax.lax.dot_general(w0b_r[...], interp, (((1,), (0,)), ((), ())),
                               preferred_element_type=jnp.float32))
    y1_r[...] = y.astype(jnp.bfloat16)

    @pl.when(jnp.logical_and(pl.program_id(0) == 0, pl.program_id(1) == 0))
    def _init():
        s_r[...] = jnp.zeros_like(s_r)
        ss_r[...] = jnp.zeros_like(ss_r)

    s_r[...] += jnp.sum(y, axis=1, keepdims=True)
    ss_r[...] += jnp.sum(y * y, axis=1, keepdims=True)


def _bn_affine(s_r, ss_r, g_r, b_r, cnt):
    mean = s_r[...] * (1.0 / cnt)
    var = ss_r[...] * (1.0 / cnt) - mean * mean
    a = g_r[...] * jax.lax.rsqrt(var + 1e-5)
    c = b_r[...] - mean * a
    return a, c


def _stage2(y1_r, s1_r, ss1_r, g0_r, b0_r, w1_r, y2_r, s_r, ss_r, *, cnt):
    a1, c1 = _bn_affine(s1_r, ss1_r, g0_r, b0_r, cnt)
    h = jnp.maximum(a1.astype(jnp.bfloat16) * y1_r[...]
                    + c1.astype(jnp.bfloat16), jnp.bfloat16(0))
    y = jax.lax.dot_general(w1_r[...], h, (((1,), (0,)), ((), ())),
                            preferred_element_type=jnp.float32)
    y2_r[...] = y.astype(jnp.bfloat16)

    @pl.when(pl.program_id(0) == 0)
    def _init():
        s_r[...] = jnp.zeros_like(s_r)
        ss_r[...] = jnp.zeros_like(ss_r)

    s_r[...] += jnp.sum(y, axis=1, keepdims=True)
    ss_r[...] += jnp.sum(y * y, axis=1, keepdims=True)


def _stage3(y2_r, s2_r, ss2_r, g1_r, b1_r, out_r, *, cnt):
    a2, c2 = _bn_affine(s2_r, ss2_r, g1_r, b1_r, cnt)
    out_r[0] = jnp.maximum(a2 * y2_r[...].astype(jnp.float32) + c2,
                           jnp.float32(0))


def kernel(xyz1, xyz2, points1, points2, W0, b0, gamma0, beta0,
           W1, b1, gamma1, beta1):
    B, _, N = xyz1.shape
    S = xyz2.shape[2]
    D = points1.shape[1]
    O0 = W0.shape[0]
    O1 = W1.shape[0]
    BN = B * N
    NB1 = 2048
    nblk1 = N // NB1
    NB2 = 4096
    NB3 = 4096
    nblk3 = N // NB3
    cnt = float(BN)

    xyz2t = jnp.transpose(xyz2, (0, 2, 1))
    g0 = gamma0[:, None]
    be0 = beta0[:, None]
    g1 = gamma1[:, None]
    be1 = beta1[:, None]

    y1, s1, ss1 = pl.pallas_call(
        _stage1,
        grid=(B, nblk1),
        in_specs=[
            pl.BlockSpec((1, 3, NB1), lambda b, n: (b, 0, n)),
            pl.BlockSpec((1, S, 3), lambda b, n: (b, 0, 0)),
            pl.BlockSpec((1, D, NB1), lambda b, n: (b, 0, n)),
            pl.BlockSpec((1, D, S), lambda b, n: (b, 0, 0)),
            pl.BlockSpec((O0, D), lambda b, n: (0, 0)),
            pl.BlockSpec((O0, D), lambda b, n: (0, 1)),
        ],
        out_specs=[
            pl.BlockSpec((O0, NB1), lambda b, n: (0, b * (N // NB1) + n)),
            pl.BlockSpec((O0, 1), lambda b, n: (0, 0)),
            pl.BlockSpec((O0, 1), lambda b, n: (0, 0)),
        ],
        out_shape=[
            jax.ShapeDtypeStruct((O0, BN), jnp.bfloat16),
            jax.ShapeDtypeStruct((O0, 1), jnp.float32),
            jax.ShapeDtypeStruct((O0, 1), jnp.float32),
        ],
    )(xyz1, xyz2t, points1, points2, W0.astype(jnp.bfloat16),
      W0.astype(jnp.bfloat16))

    y2, s2, ss2 = pl.pallas_call(
        functools.partial(_stage2, cnt=cnt),
        grid=(BN // NB2,),
        in_specs=[
            pl.BlockSpec((O0, NB2), lambda i: (0, i)),
            pl.BlockSpec((O0, 1), lambda i: (0, 0)),
            pl.BlockSpec((O0, 1), lambda i: (0, 0)),
            pl.BlockSpec((O0, 1), lambda i: (0, 0)),
            pl.BlockSpec((O0, 1), lambda i: (0, 0)),
            pl.BlockSpec((O1, O0), lambda i: (0, 0)),
        ],
        out_specs=[
            pl.BlockSpec((O1, NB2), lambda i: (0, i)),
            pl.BlockSpec((O1, 1), lambda i: (0, 0)),
            pl.BlockSpec((O1, 1), lambda i: (0, 0)),
        ],
        out_shape=[
            jax.ShapeDtypeStruct((O1, BN), jnp.bfloat16),
            jax.ShapeDtypeStruct((O1, 1), jnp.float32),
            jax.ShapeDtypeStruct((O1, 1), jnp.float32),
        ],
    )(y1, s1, ss1, g0, be0, W1.astype(jnp.bfloat16))

    out = pl.pallas_call(
        functools.partial(_stage3, cnt=cnt),
        grid=(B * nblk3,),
        in_specs=[
            pl.BlockSpec((O1, NB3), lambda i: (0, i)),
            pl.BlockSpec((O1, 1), lambda i: (0, 0)),
            pl.BlockSpec((O1, 1), lambda i: (0, 0)),
            pl.BlockSpec((O1, 1), lambda i: (0, 0)),
            pl.BlockSpec((O1, 1), lambda i: (0, 0)),
        ],
        out_specs=pl.BlockSpec((1, O1, NB3),
                               lambda i: (i // (N // NB3), 0, i % (N // NB3))),
        out_shape=jax.ShapeDtypeStruct((B, O1, N), jnp.float32),
    )(y2, s2, ss2, g1, be1)

    return out
